# Initial kernel scaffold; baseline (speedup 1.0000x reference)
import jax
import jax.numpy as jnp
from jax.experimental import pallas as pl


def kernel(x, edge_index, W1, b1, W2, b2):
    raise NotImplementedError("write your pallas kernel here")


# speedup vs baseline: 29.1057x; 29.1057x over previous
import functools

import jax
import jax.numpy as jnp
from jax import lax
from jax.experimental import pallas as pl
from jax.experimental.pallas import tpu as pltpu
from jax.experimental.pallas import tpu_sc as plsc

_N = 100000
_NP = 100096
_E = 6400000
_EP = 6422528
_EPR = _EP // 128
_GT = _EPR // 32
_NCHUNK = _GT // 16
_RZ = _NP // 16
_DEG_EDGES = (2 * _E) // 32
_DCH = 2000
_DNCH = _DEG_EDGES // _DCH

_mesh = plsc.VectorSubcoreMesh(core_axis_name="c", subcore_axis_name="s")
_sc_params = pltpu.CompilerParams(needs_layout_passes=False,
                                  use_tc_tiling_on_sc=False)


@functools.partial(
    pl.kernel,
    mesh=_mesh,
    compiler_params=_sc_params,
    out_type=jax.ShapeDtypeStruct((32 * _N,), jnp.float32),
    scratch_types=[
        pltpu.VMEM((_N,), jnp.float32),
        pltpu.VMEM((_DCH,), jnp.int32),
    ],
)
def _degrees_sc(ef_hbm, out_hbm, hist_v, idx_v):
    c = lax.axis_index("c")
    s = lax.axis_index("s")
    w = c * 16 + s
    zero16 = jnp.zeros((16,), jnp.float32)
    ones16 = jnp.ones((16,), jnp.float32)

    def zbody(i, carry):
        hist_v[pl.ds(i * 16, 16)] = zero16
        return carry

    lax.fori_loop(0, _N // 16, zbody, 0)

    ebase = w * _DEG_EDGES

    def chunk(ch, carry):
        pltpu.sync_copy(ef_hbm.at[pl.ds(ebase + ch * _DCH, _DCH)], idx_v)

        def inner(k, carry2):
            idx16 = idx_v[pl.ds(k * 16, 16)]
            plsc.addupdate_scatter(hist_v, [idx16], ones16)
            return carry2

        lax.fori_loop(0, _DCH // 16, inner, 0)
        return carry

    lax.fori_loop(0, _DNCH, chunk, 0)
    pltpu.sync_copy(hist_v, out_hbm.at[pl.ds(w * _N, _N)])


def _make_edge_pass(F):
    @functools.partial(
        pl.kernel,
        mesh=_mesh,
        compiler_params=_sc_params,
        out_type=jax.ShapeDtypeStruct((2 * _NP, F), jnp.float32),
        scratch_types=[
            pltpu.VMEM_SHARED((_NP, F), jnp.float32),
            pltpu.VMEM((16, 128), jnp.int32),
            pltpu.VMEM((16, 128), jnp.int32),
            pltpu.VMEM((128, F), jnp.float32),
            pltpu.SemaphoreType.DMA,
        ],
    )
    def pass_fn(tab_hbm, srcp_hbm, dstp_hbm, z_hbm, out_hbm,
                agg_sh, src_v, dst_v, rows_v, sem):
        c = lax.axis_index("c")
        s = lax.axis_index("s")
        w = c * 16 + s
        pltpu.sync_copy(z_hbm.at[pl.ds(s * _RZ, _RZ)],
                        agg_sh.at[pl.ds(s * _RZ, _RZ)])
        plsc.subcore_barrier()

        base_row = w * _GT

        def chunk(ch, carry):
            r0 = base_row + ch * 16
            pltpu.sync_copy(srcp_hbm.at[pl.ds(r0, 16)], src_v)
            pltpu.sync_copy(dstp_hbm.at[pl.ds(r0, 16)], dst_v)

            def grp(j, carry2):
                pltpu.async_copy(tab_hbm.at[src_v.at[j]], rows_v, sem).wait()
                pltpu.sync_copy(rows_v, agg_sh.at[dst_v.at[j]], add=True)
                return carry2

            lax.fori_loop(0, 16, grp, 0)
            return carry

        lax.fori_loop(0, _NCHUNK, chunk, 0)
        plsc.subcore_barrier()
        pltpu.sync_copy(agg_sh.at[pl.ds(s * _RZ, _RZ)],
                        out_hbm.at[pl.ds(c * _NP + s * _RZ, _RZ)])

    return pass_fn


_edge_pass8 = _make_edge_pass(8)

_BR = 1024
_GRID = (_N + _BR - 1) // _BR


def _norm(d):
    return jnp.where(d > 0, lax.rsqrt(jnp.maximum(d, 1.0)), 0.0)


def _tc1_body(x_ref, dp_ref, w1_ref, h_ref):
    ns = _norm(jnp.sum(dp_ref[...], axis=0))
    h_ref[...] = jnp.dot(x_ref[...], w1_ref[...],
                         preferred_element_type=jnp.float32,
                         precision=lax.Precision.HIGHEST) * ns[:, None]


_tc1 = pl.pallas_call(
    _tc1_body,
    grid=(_GRID,),
    in_specs=[
        pl.BlockSpec((_BR, 128), lambda i: (i, 0)),
        pl.BlockSpec((16, _BR), lambda i: (0, i)),
        pl.BlockSpec((128, 8), lambda i: (0, 0)),
    ],
    out_specs=pl.BlockSpec((_BR, 8), lambda i: (i, 0)),
    out_shape=jax.ShapeDtypeStruct((_N, 8), jnp.float32),
)


def _tc2_body(aggp_ref, dp_ref, b1_ref, u_ref):
    agg = aggp_ref[0] + aggp_ref[1]
    ns = _norm(jnp.sum(dp_ref[0:16], axis=0))
    nd = _norm(jnp.sum(dp_ref[16:32], axis=0))
    g = jnp.maximum(agg * nd[:, None] + b1_ref[...], 0.0)
    u_ref[...] = g * ns[:, None]


_tc2 = pl.pallas_call(
    _tc2_body,
    grid=(_GRID,),
    in_specs=[
        pl.BlockSpec((2, _BR, 8), lambda i: (0, i, 0)),
        pl.BlockSpec((32, _BR), lambda i: (0, i)),
        pl.BlockSpec((1, 8), lambda i: (0, 0)),
    ],
    out_specs=pl.BlockSpec((_BR, 8), lambda i: (i, 0)),
    out_shape=jax.ShapeDtypeStruct((_N, 8), jnp.float32),
)


def _tc3_body(aggp_ref, dp_ref, w2_ref, b2_ref, o_ref):
    agg = aggp_ref[0] + aggp_ref[1]
    nd = _norm(jnp.sum(dp_ref[...], axis=0))
    o_ref[...] = jnp.dot(agg, w2_ref[...],
                         preferred_element_type=jnp.float32,
                         precision=lax.Precision.HIGHEST) * nd[:, None] + b2_ref[...]


_tc3 = pl.pallas_call(
    _tc3_body,
    grid=(_GRID,),
    in_specs=[
        pl.BlockSpec((2, _BR, 8), lambda i: (0, i, 0)),
        pl.BlockSpec((16, _BR), lambda i: (1, i)),
        pl.BlockSpec((8, 1), lambda i: (0, 0)),
        pl.BlockSpec((1, 1), lambda i: (0, 0)),
    ],
    out_specs=pl.BlockSpec((_BR, 1), lambda i: (i, 0)),
    out_shape=jax.ShapeDtypeStruct((_N, 1), jnp.float32),
)


def kernel(x, edge_index, W1, b1, W2, b2):
    ei = edge_index.astype(jnp.int32)
    ef = ei.reshape(-1)
    degp = _degrees_sc(ef).reshape(32, _N)

    pad = jnp.full((_EP - _E,), _N, jnp.int32)
    srcp = jnp.concatenate([ei[0], pad]).reshape(_EPR, 128)
    dstp = jnp.concatenate([ei[1], pad]).reshape(_EPR, 128)

    h = _tc1(x, degp, W1)
    hp = jnp.concatenate([h, jnp.zeros((_NP - _N, 8), jnp.float32)], axis=0)
    z8 = jnp.zeros((_NP, 8), jnp.float32)
    aggp = _edge_pass8(hp, srcp, dstp, z8).reshape(2, _NP, 8)

    u = _tc2(aggp, degp, b1.reshape(1, 8))
    up = jnp.concatenate([u, jnp.zeros((_NP - _N, 8), jnp.float32)], axis=0)
    agg2p = _edge_pass8(up, srcp, dstp, z8).reshape(2, _NP, 8)

    return _tc3(agg2p, degp, W2, b2.reshape(1, 1))


# speedup vs baseline: 58.6803x; 2.0161x over previous
import functools

import jax
import jax.numpy as jnp
from jax import lax
from jax.experimental import pallas as pl
from jax.experimental.pallas import tpu as pltpu
from jax.experimental.pallas import tpu_sc as plsc

_N = 100000
_NP = 100096
_E = 6400000
_EP = 6422528
_EPR = _EP // 128
_GT = _EPR // 32
_NCHUNK = _GT // 16
_RZ = _NP // 16
_DEG_EDGES = (2 * _E) // 32
_DCH = 1600
_DNCH = _DEG_EDGES // _DCH

_mesh = plsc.VectorSubcoreMesh(core_axis_name="c", subcore_axis_name="s")
_sc_params = pltpu.CompilerParams(needs_layout_passes=False,
                                  use_tc_tiling_on_sc=False)


@functools.partial(
    pl.kernel,
    mesh=_mesh,
    compiler_params=_sc_params,
    out_type=jax.ShapeDtypeStruct((32 * _N,), jnp.float32),
    scratch_types=[
        pltpu.VMEM((_N,), jnp.float32),
        pltpu.VMEM((_DCH,), jnp.int32),
    ],
)
def _degrees_sc(ef_hbm, zf_hbm, out_hbm, hist_v, idx_v):
    c = lax.axis_index("c")
    s = lax.axis_index("s")
    w = c * 16 + s
    ones16 = jnp.ones((16,), jnp.float32)
    pltpu.sync_copy(zf_hbm.at[pl.ds(0, _N)], hist_v)

    ebase = w * _DEG_EDGES

    def chunk(ch, carry):
        pltpu.sync_copy(ef_hbm.at[pl.ds(ebase + ch * _DCH, _DCH)], idx_v)

        def inner(k, carry2):
            for u in range(4):
                idx16 = idx_v[pl.ds(k * 64 + u * 16, 16)]
                plsc.addupdate_scatter(hist_v, [idx16], ones16)
            return carry2

        lax.fori_loop(0, _DCH // 64, inner, 0)
        return carry

    lax.fori_loop(0, _DNCH, chunk, 0)
    pltpu.sync_copy(hist_v, out_hbm.at[pl.ds(w * _N, _N)])


_CG = 8
_NCH2 = _GT // _CG
_CROWS = _CG * 128


def _make_edge_pass(F):
    @functools.partial(
        pl.kernel,
        mesh=_mesh,
        compiler_params=_sc_params,
        out_type=jax.ShapeDtypeStruct((2 * _NP, F), jnp.float32),
        scratch_types=[
            pltpu.VMEM_SHARED((_NP, F), jnp.float32),
            pltpu.VMEM((_CG, 128), jnp.int32),
            pltpu.VMEM((_CG, 128), jnp.int32),
            pltpu.VMEM((_CG, 128), jnp.int32),
            pltpu.VMEM((_CG, 128), jnp.int32),
            pltpu.VMEM((_CROWS, F), jnp.float32),
            pltpu.VMEM((_CROWS, F), jnp.float32),
            pltpu.SemaphoreType.DMA,
            pltpu.SemaphoreType.DMA,
        ],
    )
    def pass_fn(tab_hbm, srcp_hbm, dstp_hbm, z_hbm, out_hbm,
                agg_sh, src_a, src_b, dst_a, dst_b, rows_a, rows_b,
                sem_a, sem_b):
        c = lax.axis_index("c")
        s = lax.axis_index("s")
        w = c * 16 + s
        pltpu.sync_copy(z_hbm.at[pl.ds(s * _RZ, _RZ)],
                        agg_sh.at[pl.ds(s * _RZ, _RZ)])
        plsc.subcore_barrier()

        base_row = w * _GT

        def fire(ch, src_v, rows_v, sem):
            r0 = base_row + ch * _CG
            pltpu.sync_copy(srcp_hbm.at[pl.ds(r0, _CG)], src_v)

            def f(j, carry):
                pltpu.async_copy(tab_hbm.at[src_v.at[j]],
                                 rows_v.at[pl.ds(j * 128, 128)], sem)
                return carry

            lax.fori_loop(0, _CG, f, 0)

        def drain_scatter(ch, dst_v, rows_v, sem):
            r0 = base_row + ch * _CG
            pltpu.sync_copy(dstp_hbm.at[pl.ds(r0, _CG)], dst_v)
            pltpu.make_async_copy(z_hbm.at[pl.ds(0, _CROWS)], rows_v,
                                  sem).wait()

            def sct(j, carry):
                pltpu.sync_copy(rows_v.at[pl.ds(j * 128, 128)],
                                agg_sh.at[dst_v.at[j]], add=True)
                return carry

            lax.fori_loop(0, _CG, sct, 0)

        fire(0, src_a, rows_a, sem_a)

        def body(k, carry):
            fire(2 * k + 1, src_b, rows_b, sem_b)
            drain_scatter(2 * k, dst_a, rows_a, sem_a)

            @pl.when(k < _NCH2 // 2 - 1)
            def _():
                fire(2 * k + 2, src_a, rows_a, sem_a)

            drain_scatter(2 * k + 1, dst_b, rows_b, sem_b)
            return carry

        lax.fori_loop(0, _NCH2 // 2, body, 0)
        plsc.subcore_barrier()
        pltpu.sync_copy(agg_sh.at[pl.ds(s * _RZ, _RZ)],
                        out_hbm.at[pl.ds(c * _NP + s * _RZ, _RZ)])

    return pass_fn


_edge_pass8 = _make_edge_pass(8)

_BR = 1024
_GRID = (_N + _BR - 1) // _BR


def _norm(d):
    return jnp.where(d > 0, lax.rsqrt(jnp.maximum(d, 1.0)), 0.0)


def _tc1_body(x_ref, dp_ref, w1_ref, h_ref):
    ns = _norm(jnp.sum(dp_ref[...], axis=0))
    h_ref[...] = jnp.dot(x_ref[...], w1_ref[...],
                         preferred_element_type=jnp.float32,
                         precision=lax.Precision.HIGHEST) * ns[:, None]


_tc1 = pl.pallas_call(
    _tc1_body,
    grid=(_GRID,),
    in_specs=[
        pl.BlockSpec((_BR, 128), lambda i: (i, 0)),
        pl.BlockSpec((16, _BR), lambda i: (0, i)),
        pl.BlockSpec((128, 8), lambda i: (0, 0)),
    ],
    out_specs=pl.BlockSpec((_BR, 8), lambda i: (i, 0)),
    out_shape=jax.ShapeDtypeStruct((_N, 8), jnp.float32),
)


def _tc2_body(aggp_ref, dp_ref, b1_ref, u_ref):
    agg = aggp_ref[0] + aggp_ref[1]
    ns = _norm(jnp.sum(dp_ref[0:16], axis=0))
    nd = _norm(jnp.sum(dp_ref[16:32], axis=0))
    g = jnp.maximum(agg * nd[:, None] + b1_ref[...], 0.0)
    u_ref[...] = g * ns[:, None]


_tc2 = pl.pallas_call(
    _tc2_body,
    grid=(_GRID,),
    in_specs=[
        pl.BlockSpec((2, _BR, 8), lambda i: (0, i, 0)),
        pl.BlockSpec((32, _BR), lambda i: (0, i)),
        pl.BlockSpec((1, 8), lambda i: (0, 0)),
    ],
    out_specs=pl.BlockSpec((_BR, 8), lambda i: (i, 0)),
    out_shape=jax.ShapeDtypeStruct((_N, 8), jnp.float32),
)


def _tc3_body(aggp_ref, dp_ref, w2_ref, b2_ref, o_ref):
    agg = aggp_ref[0] + aggp_ref[1]
    nd = _norm(jnp.sum(dp_ref[...], axis=0))
    o_ref[...] = jnp.dot(agg, w2_ref[...],
                         preferred_element_type=jnp.float32,
                         precision=lax.Precision.HIGHEST) * nd[:, None] + b2_ref[...]


_tc3 = pl.pallas_call(
    _tc3_body,
    grid=(_GRID,),
    in_specs=[
        pl.BlockSpec((2, _BR, 8), lambda i: (0, i, 0)),
        pl.BlockSpec((16, _BR), lambda i: (1, i)),
        pl.BlockSpec((8, 1), lambda i: (0, 0)),
        pl.BlockSpec((1, 1), lambda i: (0, 0)),
    ],
    out_specs=pl.BlockSpec((_BR, 1), lambda i: (i, 0)),
    out_shape=jax.ShapeDtypeStruct((_N, 1), jnp.float32),
)


def kernel(x, edge_index, W1, b1, W2, b2):
    ei = edge_index.astype(jnp.int32)
    ef = ei.reshape(-1)
    zf = jnp.zeros((_NP * 8,), jnp.float32)
    degp = _degrees_sc(ef, zf).reshape(32, _N)

    pad = jnp.full((_EP - _E,), _N, jnp.int32)
    srcp = jnp.concatenate([ei[0], pad]).reshape(_EPR, 128)
    dstp = jnp.concatenate([ei[1], pad]).reshape(_EPR, 128)

    h = _tc1(x, degp, W1)
    hp = jnp.concatenate([h, jnp.zeros((_NP - _N, 8), jnp.float32)], axis=0)
    z8 = zf.reshape(_NP, 8)
    aggp = _edge_pass8(hp, srcp, dstp, z8).reshape(2, _NP, 8)

    u = _tc2(aggp, degp, b1.reshape(1, 8))
    up = jnp.concatenate([u, jnp.zeros((_NP - _N, 8), jnp.float32)], axis=0)
    agg2p = _edge_pass8(up, srcp, dstp, z8).reshape(2, _NP, 8)

    return _tc3(agg2p, degp, W2, b2.reshape(1, 1))


# speedup vs baseline: 59.2223x; 1.0092x over previous
import functools

import jax
import jax.numpy as jnp
from jax import lax
from jax.experimental import pallas as pl
from jax.experimental.pallas import tpu as pltpu
from jax.experimental.pallas import tpu_sc as plsc

_N = 100000
_NP = 100096
_E = 6400000
_EP = 6422528
_EPR = _EP // 128
_GT = _EPR // 32
_NCHUNK = _GT // 16
_RZ = _NP // 16
_DEG_EDGES = (2 * _E) // 32
_DCH = 1600
_DNCH = _DEG_EDGES // _DCH

_mesh = plsc.VectorSubcoreMesh(core_axis_name="c", subcore_axis_name="s")
_sc_params = pltpu.CompilerParams(needs_layout_passes=False,
                                  use_tc_tiling_on_sc=False)


@functools.partial(
    pl.kernel,
    mesh=_mesh,
    compiler_params=_sc_params,
    out_type=jax.ShapeDtypeStruct((32 * _N,), jnp.float32),
    scratch_types=[
        pltpu.VMEM((_N,), jnp.float32),
        pltpu.VMEM((_DCH,), jnp.int32),
    ],
)
def _degrees_sc(ef_hbm, zf_hbm, out_hbm, hist_v, idx_v):
    c = lax.axis_index("c")
    s = lax.axis_index("s")
    w = c * 16 + s
    ones16 = jnp.ones((16,), jnp.float32)
    pltpu.sync_copy(zf_hbm.at[pl.ds(0, _N)], hist_v)

    ebase = w * _DEG_EDGES

    def chunk(ch, carry):
        pltpu.sync_copy(ef_hbm.at[pl.ds(ebase + ch * _DCH, _DCH)], idx_v)

        def inner(k, carry2):
            for u in range(4):
                idx16 = idx_v[pl.ds(k * 64 + u * 16, 16)]
                plsc.addupdate_scatter(hist_v, [idx16], ones16)
            return carry2

        lax.fori_loop(0, _DCH // 64, inner, 0)
        return carry

    lax.fori_loop(0, _DNCH, chunk, 0)
    pltpu.sync_copy(hist_v, out_hbm.at[pl.ds(w * _N, _N)])


_CG = 16
_NCH2 = _GT // _CG
_CROWS = _CG * 128


def _make_edge_pass(F):
    @functools.partial(
        pl.kernel,
        mesh=_mesh,
        compiler_params=_sc_params,
        out_type=jax.ShapeDtypeStruct((2 * _NP, F), jnp.float32),
        scratch_types=[
            pltpu.VMEM_SHARED((_NP, F), jnp.float32),
            pltpu.VMEM((_CG, 128), jnp.int32),
            pltpu.VMEM((_CG, 128), jnp.int32),
            pltpu.VMEM((_CG, 128), jnp.int32),
            pltpu.VMEM((_CG, 128), jnp.int32),
            pltpu.VMEM((_CROWS, F), jnp.float32),
            pltpu.VMEM((_CROWS, F), jnp.float32),
            pltpu.SemaphoreType.DMA,
            pltpu.SemaphoreType.DMA,
            pltpu.SemaphoreType.DMA,
            pltpu.SemaphoreType.DMA,
        ],
    )
    def pass_fn(tab_hbm, srcp_hbm, dstp_hbm, z_hbm, out_hbm,
                agg_sh, src_a, src_b, dst_a, dst_b, rows_a, rows_b,
                gsem_a, gsem_b, ssem_a, ssem_b):
        c = lax.axis_index("c")
        s = lax.axis_index("s")
        w = c * 16 + s
        pltpu.sync_copy(z_hbm.at[pl.ds(s * _RZ, _RZ)],
                        agg_sh.at[pl.ds(s * _RZ, _RZ)])
        plsc.subcore_barrier()

        base_row = w * _GT

        def fire_g(ch, src_v, rows_v, sem):
            r0 = base_row + ch * _CG
            pltpu.sync_copy(srcp_hbm.at[pl.ds(r0, _CG)], src_v)

            def f(j, carry):
                pltpu.async_copy(tab_hbm.at[src_v.at[j]],
                                 rows_v.at[pl.ds(j * 128, 128)], sem)
                return carry

            lax.fori_loop(0, _CG, f, 0)

        def wait_bytes(rows_v, sem):
            pltpu.make_async_copy(z_hbm.at[pl.ds(0, _CROWS)], rows_v,
                                  sem).wait()

        def issue_s(ch, dst_v, rows_v, sem):
            r0 = base_row + ch * _CG
            pltpu.sync_copy(dstp_hbm.at[pl.ds(r0, _CG)], dst_v)

            def sct(j, carry):
                pltpu.async_copy(rows_v.at[pl.ds(j * 128, 128)],
                                 agg_sh.at[dst_v.at[j]], sem, add=True)
                return carry

            lax.fori_loop(0, _CG, sct, 0)

        fire_g(0, src_a, rows_a, gsem_a)
        wait_bytes(rows_a, gsem_a)
        fire_g(1, src_b, rows_b, gsem_b)
        issue_s(0, dst_a, rows_a, ssem_a)

        def body(k, carry):
            wait_bytes(rows_b, gsem_b)
            wait_bytes(rows_a, ssem_a)

            @pl.when(2 * k + 2 < _NCH2)
            def _():
                fire_g(2 * k + 2, src_a, rows_a, gsem_a)

            issue_s(2 * k + 1, dst_b, rows_b, ssem_b)

            @pl.when(2 * k + 2 < _NCH2)
            def _():
                wait_bytes(rows_a, gsem_a)
                wait_bytes(rows_b, ssem_b)

                @pl.when(2 * k + 3 < _NCH2)
                def _():
                    fire_g(2 * k + 3, src_b, rows_b, gsem_b)

                issue_s(2 * k + 2, dst_a, rows_a, ssem_a)

            return carry

        lax.fori_loop(0, _NCH2 // 2, body, 0)
        wait_bytes(rows_b, ssem_b)
        plsc.subcore_barrier()
        pltpu.sync_copy(agg_sh.at[pl.ds(s * _RZ, _RZ)],
                        out_hbm.at[pl.ds(c * _NP + s * _RZ, _RZ)])

    return pass_fn


_edge_pass8 = _make_edge_pass(8)

_BR = 1024
_GRID = (_N + _BR - 1) // _BR


def _norm(d):
    return jnp.where(d > 0, lax.rsqrt(jnp.maximum(d, 1.0)), 0.0)


def _tc1a_body(x_ref, w1_ref, xw_ref):
    xw_ref[...] = jnp.dot(x_ref[...], w1_ref[...],
                          preferred_element_type=jnp.float32,
                          precision=lax.Precision.HIGHEST)


_tc1a = pl.pallas_call(
    _tc1a_body,
    grid=(_GRID,),
    in_specs=[
        pl.BlockSpec((_BR, 128), lambda i: (i, 0)),
        pl.BlockSpec((128, 8), lambda i: (0, 0)),
    ],
    out_specs=pl.BlockSpec((_BR, 8), lambda i: (i, 0)),
    out_shape=jax.ShapeDtypeStruct((_N, 8), jnp.float32),
)


def _tc1b_body(xw_ref, dp_ref, h_ref):
    ns = _norm(jnp.sum(dp_ref[...], axis=0))
    h_ref[...] = xw_ref[...] * ns[:, None]


_tc1b = pl.pallas_call(
    _tc1b_body,
    grid=(_GRID,),
    in_specs=[
        pl.BlockSpec((_BR, 8), lambda i: (i, 0)),
        pl.BlockSpec((16, _BR), lambda i: (0, i)),
    ],
    out_specs=pl.BlockSpec((_BR, 8), lambda i: (i, 0)),
    out_shape=jax.ShapeDtypeStruct((_N, 8), jnp.float32),
)


def _tc2_body(aggp_ref, dp_ref, b1_ref, u_ref):
    agg = aggp_ref[0] + aggp_ref[1]
    ns = _norm(jnp.sum(dp_ref[0:16], axis=0))
    nd = _norm(jnp.sum(dp_ref[16:32], axis=0))
    g = jnp.maximum(agg * nd[:, None] + b1_ref[...], 0.0)
    u_ref[...] = g * ns[:, None]


_tc2 = pl.pallas_call(
    _tc2_body,
    grid=(_GRID,),
    in_specs=[
        pl.BlockSpec((2, _BR, 8), lambda i: (0, i, 0)),
        pl.BlockSpec((32, _BR), lambda i: (0, i)),
        pl.BlockSpec((1, 8), lambda i: (0, 0)),
    ],
    out_specs=pl.BlockSpec((_BR, 8), lambda i: (i, 0)),
    out_shape=jax.ShapeDtypeStruct((_N, 8), jnp.float32),
)


def _tc3_body(aggp_ref, dp_ref, w2_ref, b2_ref, o_ref):
    agg = aggp_ref[0] + aggp_ref[1]
    nd = _norm(jnp.sum(dp_ref[...], axis=0))
    o_ref[...] = jnp.dot(agg, w2_ref[...],
                         preferred_element_type=jnp.float32,
                         precision=lax.Precision.HIGHEST) * nd[:, None] + b2_ref[...]


_tc3 = pl.pallas_call(
    _tc3_body,
    grid=(_GRID,),
    in_specs=[
        pl.BlockSpec((2, _BR, 8), lambda i: (0, i, 0)),
        pl.BlockSpec((16, _BR), lambda i: (1, i)),
        pl.BlockSpec((8, 1), lambda i: (0, 0)),
        pl.BlockSpec((1, 1), lambda i: (0, 0)),
    ],
    out_specs=pl.BlockSpec((_BR, 1), lambda i: (i, 0)),
    out_shape=jax.ShapeDtypeStruct((_N, 1), jnp.float32),
)


def kernel(x, edge_index, W1, b1, W2, b2):
    ei = edge_index.astype(jnp.int32)
    ef = ei.reshape(-1)
    zf = jnp.zeros((_NP * 8,), jnp.float32)
    degp = _degrees_sc(ef, zf).reshape(32, _N)

    pad = jnp.full((_EP - _E,), _N, jnp.int32)
    srcp = jnp.concatenate([ei[0], pad]).reshape(_EPR, 128)
    dstp = jnp.concatenate([ei[1], pad]).reshape(_EPR, 128)

    xw = _tc1a(x, W1)
    h = _tc1b(xw, degp)
    hp = jnp.concatenate([h, jnp.zeros((_NP - _N, 8), jnp.float32)], axis=0)
    z8 = zf.reshape(_NP, 8)
    aggp = _edge_pass8(hp, srcp, dstp, z8).reshape(2, _NP, 8)

    u = _tc2(aggp, degp, b1.reshape(1, 8))
    up = jnp.concatenate([u, jnp.zeros((_NP - _N, 8), jnp.float32)], axis=0)
    agg2p = _edge_pass8(up, srcp, dstp, z8).reshape(2, _NP, 8)

    return _tc3(agg2p, degp, W2, b2.reshape(1, 1))


# speedup vs baseline: 72.9559x; 1.2319x over previous
import functools

import jax
import jax.numpy as jnp
from jax import lax
from jax.experimental import pallas as pl
from jax.experimental.pallas import tpu as pltpu
from jax.experimental.pallas import tpu_sc as plsc

_N = 100000
_NP = 100096
_E = 6400000
_EPR = _E // 128
_GC = _EPR // 2
_CG = 16
_M = 97
_RZ = _NP // 16
_DEG_EDGES = (2 * _E) // 32
_DCH = 1600
_DNCH = _DEG_EDGES // _DCH

_mesh = plsc.VectorSubcoreMesh(core_axis_name="c", subcore_axis_name="s")
_sc_params = pltpu.CompilerParams(needs_layout_passes=False,
                                  use_tc_tiling_on_sc=False)


@functools.partial(
    pl.kernel,
    mesh=_mesh,
    compiler_params=_sc_params,
    out_type=jax.ShapeDtypeStruct((32 * _N,), jnp.float32),
    scratch_types=[
        pltpu.VMEM((_N,), jnp.float32),
        pltpu.VMEM((_DCH,), jnp.int32),
        pltpu.VMEM((_DCH,), jnp.int32),
        pltpu.SemaphoreType.DMA,
        pltpu.SemaphoreType.DMA,
    ],
)
def _degrees_sc(ef_hbm, zf_hbm, out_hbm, hist_v, idx_a, idx_b, sem_a, sem_b):
    c = lax.axis_index("c")
    s = lax.axis_index("s")
    w = c * 16 + s
    ones16 = jnp.ones((16,), jnp.float32)
    pltpu.sync_copy(zf_hbm.at[pl.ds(0, _N)], hist_v)

    ebase = w * _DEG_EDGES

    def stage(ch, idx_v, sem):
        pltpu.async_copy(ef_hbm.at[pl.ds(ebase + ch * _DCH, _DCH)], idx_v,
                         sem)

    def wait_stage(idx_v, sem):
        pltpu.make_async_copy(ef_hbm.at[pl.ds(0, _DCH)], idx_v, sem).wait()

    def hist(idx_v):
        @plsc.parallel_loop(0, _DCH, 16, unroll=8)
        def _(i):
            plsc.addupdate_scatter(hist_v, [idx_v[pl.ds(i, 16)]], ones16)

    stage(0, idx_a, sem_a)

    def body(ch, carry):
        even = lax.rem(ch, 2) == 0

        def half(idx_v, sem, o_idx, o_sem):
            wait_stage(idx_v, sem)

            @pl.when(ch + 1 < _DNCH)
            def _():
                stage(ch + 1, o_idx, o_sem)

            hist(idx_v)

        @pl.when(even)
        def _():
            half(idx_a, sem_a, idx_b, sem_b)

        @pl.when(jnp.logical_not(even))
        def _():
            half(idx_b, sem_b, idx_a, sem_a)

        return carry

    lax.fori_loop(0, _DNCH, body, 0)
    pltpu.sync_copy(hist_v, out_hbm.at[pl.ds(w * _N, _N)])


def _make_edge_pass(F):
    @functools.partial(
        pl.kernel,
        mesh=_mesh,
        compiler_params=_sc_params,
        out_type=jax.ShapeDtypeStruct((2 * _NP, F), jnp.float32),
        scratch_types=[
            pltpu.VMEM_SHARED((_NP, F), jnp.float32),
            pltpu.VMEM((_CG, 128), jnp.int32),
            pltpu.VMEM((_CG, 128), jnp.int32),
            pltpu.VMEM((_CG, 128), jnp.int32),
            pltpu.VMEM((_CG, 128), jnp.int32),
            pltpu.VMEM((_CG * 128, F), jnp.float32),
            pltpu.VMEM((_CG * 128, F), jnp.float32),
            pltpu.SemaphoreType.DMA,
            pltpu.SemaphoreType.DMA,
            pltpu.SemaphoreType.DMA,
            pltpu.SemaphoreType.DMA,
        ],
    )
    def pass_fn(tab_hbm, srcp_hbm, dstp_hbm, z_hbm, out_hbm,
                agg_sh, src_a, src_b, dst_a, dst_b, rows_a, rows_b,
                gsem_a, gsem_b, ssem_a, ssem_b):
        c = lax.axis_index("c")
        s = lax.axis_index("s")
        pltpu.sync_copy(z_hbm.at[pl.ds(s * _RZ, _RZ)],
                        agg_sh.at[pl.ds(s * _RZ, _RZ)])
        plsc.subcore_barrier()

        g0 = c * _GC + s * 1562 + jnp.minimum(s, 8)
        ng = jnp.where(s < 8, 1563, 1562)

        def fire_g(ch, src_v, rows_v, sem):
            r0 = g0 + ch * _CG
            pltpu.sync_copy(srcp_hbm.at[pl.ds(r0, _CG)], src_v)

            @plsc.parallel_loop(0, _CG, 1, unroll=4)
            def _(j):
                pltpu.async_copy(tab_hbm.at[src_v.at[j]],
                                 rows_v.at[pl.ds(j * 128, 128)], sem)

        def wait_bytes(rows_v, sem):
            pltpu.make_async_copy(z_hbm.at[pl.ds(0, _CG * 128)], rows_v,
                                  sem).wait()

        def issue_s(ch, dst_v, rows_v, sem):
            r0 = g0 + ch * _CG
            pltpu.sync_copy(dstp_hbm.at[pl.ds(r0, _CG)], dst_v)

            @plsc.parallel_loop(0, _CG, 1, unroll=4)
            def _(j):
                pltpu.async_copy(rows_v.at[pl.ds(j * 128, 128)],
                                 agg_sh.at[dst_v.at[j]], sem, add=True)

        fire_g(0, src_a, rows_a, gsem_a)

        def body(ch, carry):
            even = lax.rem(ch, 2) == 0

            def half(src_v, dst_v, rows_v, gsem, ssem,
                     o_src, o_dst, o_rows, o_gsem, o_ssem):
                wait_bytes(rows_v, gsem)

                @pl.when(ch + 1 < _M)
                def _():
                    @pl.when(ch > 0)
                    def _():
                        wait_bytes(o_rows, o_ssem)

                    fire_g(ch + 1, o_src, o_rows, o_gsem)

                issue_s(ch, dst_v, rows_v, ssem)

            @pl.when(even)
            def _():
                half(src_a, dst_a, rows_a, gsem_a, ssem_a,
                     src_b, dst_b, rows_b, gsem_b, ssem_b)

            @pl.when(jnp.logical_not(even))
            def _():
                half(src_b, dst_b, rows_b, gsem_b, ssem_b,
                     src_a, dst_a, rows_a, gsem_a, ssem_a)

            return carry

        lax.fori_loop(0, _M, body, 0)
        wait_bytes(rows_b, ssem_b)
        wait_bytes(rows_a, ssem_a)

        def tail(g, carry):
            pltpu.sync_copy(srcp_hbm.at[pl.ds(g, 1)], src_a.at[pl.ds(0, 1)])
            pltpu.async_copy(tab_hbm.at[src_a.at[0]],
                             rows_a.at[pl.ds(0, 128)], gsem_a).wait()
            pltpu.sync_copy(dstp_hbm.at[pl.ds(g, 1)], dst_a.at[pl.ds(0, 1)])
            pltpu.sync_copy(rows_a.at[pl.ds(0, 128)],
                            agg_sh.at[dst_a.at[0]], add=True)
            return carry

        lax.fori_loop(g0 + _M * _CG, g0 + ng, tail, 0)
        plsc.subcore_barrier()
        pltpu.sync_copy(agg_sh.at[pl.ds(s * _RZ, _RZ)],
                        out_hbm.at[pl.ds(c * _NP + s * _RZ, _RZ)])

    return pass_fn


_edge_pass8 = _make_edge_pass(8)

_BR = 1024
_GRID = (_N + _BR - 1) // _BR


def _norm(d):
    return jnp.where(d > 0, lax.rsqrt(jnp.maximum(d, 1.0)), 0.0)


def _tc1a_body(x_ref, w1_ref, xw_ref):
    xw_ref[...] = jnp.dot(x_ref[...], w1_ref[...],
                          preferred_element_type=jnp.float32,
                          precision=lax.Precision.HIGHEST)


_tc1a = pl.pallas_call(
    _tc1a_body,
    grid=(_GRID,),
    in_specs=[
        pl.BlockSpec((_BR, 128), lambda i: (i, 0)),
        pl.BlockSpec((128, 8), lambda i: (0, 0)),
    ],
    out_specs=pl.BlockSpec((_BR, 8), lambda i: (i, 0)),
    out_shape=jax.ShapeDtypeStruct((_N, 8), jnp.float32),
)


def _tc1b_body(xw_ref, dp_ref, h_ref):
    ns = _norm(jnp.sum(dp_ref[...], axis=0))
    h_ref[...] = xw_ref[...] * ns[:, None]


_tc1b = pl.pallas_call(
    _tc1b_body,
    grid=(_GRID,),
    in_specs=[
        pl.BlockSpec((_BR, 8), lambda i: (i, 0)),
        pl.BlockSpec((16, _BR), lambda i: (0, i)),
    ],
    out_specs=pl.BlockSpec((_BR, 8), lambda i: (i, 0)),
    out_shape=jax.ShapeDtypeStruct((_NP, 8), jnp.float32),
)


def _tc2_body(aggp_ref, dp_ref, b1_ref, u_ref):
    agg = aggp_ref[0] + aggp_ref[1]
    ns = _norm(jnp.sum(dp_ref[0:16], axis=0))
    nd = _norm(jnp.sum(dp_ref[16:32], axis=0))
    g = jnp.maximum(agg * nd[:, None] + b1_ref[...], 0.0)
    u_ref[...] = g * ns[:, None]


_tc2 = pl.pallas_call(
    _tc2_body,
    grid=(_GRID,),
    in_specs=[
        pl.BlockSpec((2, _BR, 8), lambda i: (0, i, 0)),
        pl.BlockSpec((32, _BR), lambda i: (0, i)),
        pl.BlockSpec((1, 8), lambda i: (0, 0)),
    ],
    out_specs=pl.BlockSpec((_BR, 8), lambda i: (i, 0)),
    out_shape=jax.ShapeDtypeStruct((_NP, 8), jnp.float32),
)


def _tc3_body(aggp_ref, dp_ref, w2_ref, b2_ref, o_ref):
    agg = aggp_ref[0] + aggp_ref[1]
    nd = _norm(jnp.sum(dp_ref[...], axis=0))
    o_ref[...] = jnp.dot(agg, w2_ref[...],
                         preferred_element_type=jnp.float32,
                         precision=lax.Precision.HIGHEST) * nd[:, None] + b2_ref[...]


_tc3 = pl.pallas_call(
    _tc3_body,
    grid=(_GRID,),
    in_specs=[
        pl.BlockSpec((2, _BR, 8), lambda i: (0, i, 0)),
        pl.BlockSpec((16, _BR), lambda i: (1, i)),
        pl.BlockSpec((8, 1), lambda i: (0, 0)),
        pl.BlockSpec((1, 1), lambda i: (0, 0)),
    ],
    out_specs=pl.BlockSpec((_BR, 1), lambda i: (i, 0)),
    out_shape=jax.ShapeDtypeStruct((_N, 1), jnp.float32),
)


def kernel(x, edge_index, W1, b1, W2, b2):
    ei = edge_index.astype(jnp.int32)
    ef = ei.reshape(-1)
    zf = jnp.zeros((_NP * 8,), jnp.float32)
    degp = _degrees_sc(ef, zf).reshape(32, _N)

    srcp = ei[0].reshape(_EPR, 128)
    dstp = ei[1].reshape(_EPR, 128)
    z8 = zf.reshape(_NP, 8)

    xw = _tc1a(x, W1)
    hp = _tc1b(xw, degp)
    aggp = _edge_pass8(hp, srcp, dstp, z8).reshape(2, _NP, 8)

    up = _tc2(aggp, degp, b1.reshape(1, 8))
    agg2p = _edge_pass8(up, srcp, dstp, z8).reshape(2, _NP, 8)

    return _tc3(agg2p, degp, W2, b2.reshape(1, 1))


# speedup vs baseline: 78.9442x; 1.0821x over previous
import functools

import jax
import jax.numpy as jnp
from jax import lax
from jax.experimental import pallas as pl
from jax.experimental.pallas import tpu as pltpu
from jax.experimental.pallas import tpu_sc as plsc

_N = 100000
_NP = 100096
_E = 6400000
_EPR = _E // 128
_GC = _EPR // 2
_CG = 16
_M = 97
_RZ = _NP // 16
_DEG_EDGES = (2 * _E) // 32
_DCH = 1600
_DNCH = _DEG_EDGES // _DCH

_mesh = plsc.VectorSubcoreMesh(core_axis_name="c", subcore_axis_name="s")
_sc_params = pltpu.CompilerParams(needs_layout_passes=False,
                                  use_tc_tiling_on_sc=False)


@functools.partial(
    pl.kernel,
    mesh=_mesh,
    compiler_params=_sc_params,
    out_type=jax.ShapeDtypeStruct((32 * _N,), jnp.float32),
    scratch_types=[
        pltpu.VMEM((_N,), jnp.float32),
        pltpu.VMEM((_DCH,), jnp.int32),
        pltpu.VMEM((_DCH,), jnp.int32),
        pltpu.SemaphoreType.DMA,
        pltpu.SemaphoreType.DMA,
    ],
)
def _degrees_sc(ef_hbm, zf_hbm, out_hbm, hist_v, idx_a, idx_b, sem_a, sem_b):
    c = lax.axis_index("c")
    s = lax.axis_index("s")
    w = c * 16 + s
    ones16 = jnp.ones((16,), jnp.float32)
    pltpu.sync_copy(zf_hbm.at[pl.ds(0, _N)], hist_v)

    ebase = w * _DEG_EDGES

    def stage(ch, idx_v, sem):
        pltpu.async_copy(ef_hbm.at[pl.ds(ebase + ch * _DCH, _DCH)], idx_v,
                         sem)

    def wait_stage(idx_v, sem):
        pltpu.make_async_copy(ef_hbm.at[pl.ds(0, _DCH)], idx_v, sem).wait()

    def hist(idx_v):
        @plsc.parallel_loop(0, _DCH, 16, unroll=8)
        def _(i):
            plsc.addupdate_scatter(hist_v, [idx_v[pl.ds(i, 16)]], ones16)

    stage(0, idx_a, sem_a)

    def body(ch, carry):
        even = lax.rem(ch, 2) == 0

        def half(idx_v, sem, o_idx, o_sem):
            wait_stage(idx_v, sem)

            @pl.when(ch + 1 < _DNCH)
            def _():
                stage(ch + 1, o_idx, o_sem)

            hist(idx_v)

        @pl.when(even)
        def _():
            half(idx_a, sem_a, idx_b, sem_b)

        @pl.when(jnp.logical_not(even))
        def _():
            half(idx_b, sem_b, idx_a, sem_a)

        return carry

    lax.fori_loop(0, _DNCH, body, 0)
    pltpu.sync_copy(hist_v, out_hbm.at[pl.ds(w * _N, _N)])


def _make_edge_pass(F):
    @functools.partial(
        pl.kernel,
        mesh=_mesh,
        compiler_params=_sc_params,
        out_type=jax.ShapeDtypeStruct((2 * _NP, F), jnp.float32),
        scratch_types=[
            pltpu.VMEM_SHARED((_NP, F), jnp.float32),
            pltpu.VMEM((_CG, 128), jnp.int32),
            pltpu.VMEM((_CG, 128), jnp.int32),
            pltpu.VMEM((_CG, 128), jnp.int32),
            pltpu.VMEM((_CG, 128), jnp.int32),
            pltpu.VMEM((_CG * 128, F), jnp.float32),
            pltpu.VMEM((_CG * 128, F), jnp.float32),
            pltpu.SemaphoreType.DMA,
            pltpu.SemaphoreType.DMA,
            pltpu.SemaphoreType.DMA,
            pltpu.SemaphoreType.DMA,
        ],
    )
    def pass_fn(tab_hbm, srcp_hbm, dstp_hbm, z_hbm, out_hbm,
                agg_sh, src_a, src_b, dst_a, dst_b, rows_a, rows_b,
                gsem_a, gsem_b, ssem_a, ssem_b):
        c = lax.axis_index("c")
        s = lax.axis_index("s")
        pltpu.sync_copy(z_hbm.at[pl.ds(s * _RZ, _RZ)],
                        agg_sh.at[pl.ds(s * _RZ, _RZ)])
        plsc.subcore_barrier()

        g0 = c * _GC + s * 1562 + jnp.minimum(s, 8)
        ng = jnp.where(s < 8, 1563, 1562)

        def fire_g(ch, src_v, rows_v, sem):
            r0 = g0 + ch * _CG
            pltpu.sync_copy(srcp_hbm.at[pl.ds(r0, _CG)], src_v)

            @plsc.parallel_loop(0, _CG, 1, unroll=4)
            def _(j):
                pltpu.async_copy(tab_hbm.at[src_v.at[j]],
                                 rows_v.at[pl.ds(j * 128, 128)], sem)

        def wait_bytes(rows_v, sem):
            pltpu.make_async_copy(z_hbm.at[pl.ds(0, _CG * 128)], rows_v,
                                  sem).wait()

        def issue_s(ch, dst_v, rows_v, sem):
            r0 = g0 + ch * _CG
            pltpu.sync_copy(dstp_hbm.at[pl.ds(r0, _CG)], dst_v)

            @plsc.parallel_loop(0, _CG, 1, unroll=4)
            def _(j):
                pltpu.async_copy(rows_v.at[pl.ds(j * 128, 128)],
                                 agg_sh.at[dst_v.at[j]], sem, add=True)

        fire_g(0, src_a, rows_a, gsem_a)

        def body(ch, carry):
            even = lax.rem(ch, 2) == 0

            def half(src_v, dst_v, rows_v, gsem, ssem,
                     o_src, o_dst, o_rows, o_gsem, o_ssem):
                wait_bytes(rows_v, gsem)

                @pl.when(ch + 1 < _M)
                def _():
                    @pl.when(ch > 0)
                    def _():
                        wait_bytes(o_rows, o_ssem)

                    fire_g(ch + 1, o_src, o_rows, o_gsem)

                issue_s(ch, dst_v, rows_v, ssem)

            @pl.when(even)
            def _():
                half(src_a, dst_a, rows_a, gsem_a, ssem_a,
                     src_b, dst_b, rows_b, gsem_b, ssem_b)

            @pl.when(jnp.logical_not(even))
            def _():
                half(src_b, dst_b, rows_b, gsem_b, ssem_b,
                     src_a, dst_a, rows_a, gsem_a, ssem_a)

            return carry

        lax.fori_loop(0, _M, body, 0)
        wait_bytes(rows_b, ssem_b)
        wait_bytes(rows_a, ssem_a)

        def tail(g, carry):
            pltpu.sync_copy(srcp_hbm.at[pl.ds(g, 1)], src_a.at[pl.ds(0, 1)])
            pltpu.async_copy(tab_hbm.at[src_a.at[0]],
                             rows_a.at[pl.ds(0, 128)], gsem_a).wait()
            pltpu.sync_copy(dstp_hbm.at[pl.ds(g, 1)], dst_a.at[pl.ds(0, 1)])
            pltpu.sync_copy(rows_a.at[pl.ds(0, 128)],
                            agg_sh.at[dst_a.at[0]], add=True)
            return carry

        lax.fori_loop(g0 + _M * _CG, g0 + ng, tail, 0)
        plsc.subcore_barrier()
        pltpu.sync_copy(agg_sh.at[pl.ds(s * _RZ, _RZ)],
                        out_hbm.at[pl.ds(c * _NP + s * _RZ, _RZ)])

    return pass_fn


_edge_pass8 = _make_edge_pass(8)

_BR = 8192
_GRID = (_N + _BR - 1) // _BR


def _norm(d):
    return jnp.where(d > 0, lax.rsqrt(jnp.maximum(d, 1.0)), 0.0)


def _tcn_body(dp_ref, n_ref):
    n_ref[0:1] = _norm(jnp.sum(dp_ref[0:16], axis=0))[None]
    n_ref[1:2] = _norm(jnp.sum(dp_ref[16:32], axis=0))[None]


_tcn = pl.pallas_call(
    _tcn_body,
    grid=(_GRID,),
    in_specs=[pl.BlockSpec((32, _BR), lambda i: (0, i))],
    out_specs=pl.BlockSpec((2, _BR), lambda i: (0, i)),
    out_shape=jax.ShapeDtypeStruct((2, _NP), jnp.float32),
)


def _tc1a_body(x_ref, w1_ref, xw_ref):
    xw_ref[...] = jnp.dot(x_ref[...], w1_ref[...],
                          preferred_element_type=jnp.float32,
                          precision=lax.Precision.HIGHEST)


_tc1a = pl.pallas_call(
    _tc1a_body,
    grid=(_GRID,),
    in_specs=[
        pl.BlockSpec((_BR, 128), lambda i: (i, 0)),
        pl.BlockSpec((128, 8), lambda i: (0, 0)),
    ],
    out_specs=pl.BlockSpec((_BR, 8), lambda i: (i, 0)),
    out_shape=jax.ShapeDtypeStruct((_N, 8), jnp.float32),
)


def _tc1b_body(xw_ref, n_ref, h_ref):
    h_ref[...] = xw_ref[...] * n_ref[0][:, None]


_tc1b = pl.pallas_call(
    _tc1b_body,
    grid=(_GRID,),
    in_specs=[
        pl.BlockSpec((_BR, 8), lambda i: (i, 0)),
        pl.BlockSpec((2, _BR), lambda i: (0, i)),
    ],
    out_specs=pl.BlockSpec((_BR, 8), lambda i: (i, 0)),
    out_shape=jax.ShapeDtypeStruct((_NP, 8), jnp.float32),
)


def _tc2_body(aggp_ref, n_ref, b1_ref, u_ref):
    agg = aggp_ref[0] + aggp_ref[1]
    g = jnp.maximum(agg * n_ref[1][:, None] + b1_ref[...], 0.0)
    u_ref[...] = g * n_ref[0][:, None]


_tc2 = pl.pallas_call(
    _tc2_body,
    grid=(_GRID,),
    in_specs=[
        pl.BlockSpec((2, _BR, 8), lambda i: (0, i, 0)),
        pl.BlockSpec((2, _BR), lambda i: (0, i)),
        pl.BlockSpec((1, 8), lambda i: (0, 0)),
    ],
    out_specs=pl.BlockSpec((_BR, 8), lambda i: (i, 0)),
    out_shape=jax.ShapeDtypeStruct((_NP, 8), jnp.float32),
)


def _tc3_body(aggp_ref, n_ref, w2_ref, b2_ref, o_ref):
    agg = aggp_ref[0] + aggp_ref[1]
    o_ref[...] = jnp.dot(agg, w2_ref[...],
                         preferred_element_type=jnp.float32,
                         precision=lax.Precision.HIGHEST) * n_ref[1][:, None] + b2_ref[...]


_tc3 = pl.pallas_call(
    _tc3_body,
    grid=(_GRID,),
    in_specs=[
        pl.BlockSpec((2, _BR, 8), lambda i: (0, i, 0)),
        pl.BlockSpec((2, _BR), lambda i: (0, i)),
        pl.BlockSpec((8, 1), lambda i: (0, 0)),
        pl.BlockSpec((1, 1), lambda i: (0, 0)),
    ],
    out_specs=pl.BlockSpec((_BR, 1), lambda i: (i, 0)),
    out_shape=jax.ShapeDtypeStruct((_N, 1), jnp.float32),
)


def kernel(x, edge_index, W1, b1, W2, b2):
    ei = edge_index.astype(jnp.int32)
    ef = ei.reshape(-1)
    zf = jnp.zeros((_NP * 8,), jnp.float32)
    degp = _degrees_sc(ef, zf).reshape(32, _N)

    srcp = ei[0].reshape(_EPR, 128)
    dstp = ei[1].reshape(_EPR, 128)
    z8 = zf.reshape(_NP, 8)

    xw = _tc1a(x, W1)
    nrm = _tcn(degp)
    hp = _tc1b(xw, nrm)
    aggp = _edge_pass8(hp, srcp, dstp, z8).reshape(2, _NP, 8)

    up = _tc2(aggp, nrm, b1.reshape(1, 8))
    agg2p = _edge_pass8(up, srcp, dstp, z8).reshape(2, _NP, 8)

    return _tc3(agg2p, nrm, W2, b2.reshape(1, 1))


# speedup vs baseline: 79.0458x; 1.0013x over previous
import functools

import jax
import jax.numpy as jnp
from jax import lax
from jax.experimental import pallas as pl
from jax.experimental.pallas import tpu as pltpu
from jax.experimental.pallas import tpu_sc as plsc

_N = 100000
_NP = 100096
_E = 6400000
_EPR = _E // 128
_GC = _EPR // 2
_CG = 16
_M = 97
_RZ = _NP // 16
_DEG_EDGES = (2 * _E) // 32
_DCH = 1600
_DNCH = _DEG_EDGES // _DCH

_mesh = plsc.VectorSubcoreMesh(core_axis_name="c", subcore_axis_name="s")
_sc_params = pltpu.CompilerParams(needs_layout_passes=False,
                                  use_tc_tiling_on_sc=False)


@functools.partial(
    pl.kernel,
    mesh=_mesh,
    compiler_params=_sc_params,
    out_type=jax.ShapeDtypeStruct((32 * _N,), jnp.float32),
    scratch_types=[
        pltpu.VMEM((_N,), jnp.float32),
        pltpu.VMEM((_DCH,), jnp.int32),
        pltpu.VMEM((_DCH,), jnp.int32),
        pltpu.SemaphoreType.DMA,
        pltpu.SemaphoreType.DMA,
    ],
)
def _degrees_sc(ef_hbm, zf_hbm, out_hbm, hist_v, idx_a, idx_b, sem_a, sem_b):
    c = lax.axis_index("c")
    s = lax.axis_index("s")
    w = c * 16 + s
    ones16 = jnp.ones((16,), jnp.float32)
    pltpu.sync_copy(zf_hbm.at[pl.ds(0, _N)], hist_v)

    ebase = w * _DEG_EDGES

    def stage(ch, idx_v, sem):
        pltpu.async_copy(ef_hbm.at[pl.ds(ebase + ch * _DCH, _DCH)], idx_v,
                         sem)

    def wait_stage(idx_v, sem):
        pltpu.make_async_copy(ef_hbm.at[pl.ds(0, _DCH)], idx_v, sem).wait()

    def hist(idx_v):
        @plsc.parallel_loop(0, _DCH, 16, unroll=16)
        def _(i):
            plsc.addupdate_scatter(hist_v, [idx_v[pl.ds(i, 16)]], ones16)

    stage(0, idx_a, sem_a)

    def body(ch, carry):
        even = lax.rem(ch, 2) == 0

        def half(idx_v, sem, o_idx, o_sem):
            wait_stage(idx_v, sem)

            @pl.when(ch + 1 < _DNCH)
            def _():
                stage(ch + 1, o_idx, o_sem)

            hist(idx_v)

        @pl.when(even)
        def _():
            half(idx_a, sem_a, idx_b, sem_b)

        @pl.when(jnp.logical_not(even))
        def _():
            half(idx_b, sem_b, idx_a, sem_a)

        return carry

    lax.fori_loop(0, _DNCH, body, 0)
    pltpu.sync_copy(hist_v, out_hbm.at[pl.ds(w * _N, _N)])


def _make_edge_pass(F):
    @functools.partial(
        pl.kernel,
        mesh=_mesh,
        compiler_params=_sc_params,
        out_type=jax.ShapeDtypeStruct((2 * _NP, F), jnp.float32),
        scratch_types=[
            pltpu.VMEM_SHARED((_NP, F), jnp.float32),
            pltpu.VMEM((_CG, 128), jnp.int32),
            pltpu.VMEM((_CG, 128), jnp.int32),
            pltpu.VMEM((_CG, 128), jnp.int32),
            pltpu.VMEM((_CG, 128), jnp.int32),
            pltpu.VMEM((_CG * 128, F), jnp.float32),
            pltpu.VMEM((_CG * 128, F), jnp.float32),
            pltpu.SemaphoreType.DMA,
            pltpu.SemaphoreType.DMA,
            pltpu.SemaphoreType.DMA,
            pltpu.SemaphoreType.DMA,
        ],
    )
    def pass_fn(tab_hbm, srcp_hbm, dstp_hbm, z_hbm, out_hbm,
                agg_sh, src_a, src_b, dst_a, dst_b, rows_a, rows_b,
                gsem_a, gsem_b, ssem_a, ssem_b):
        c = lax.axis_index("c")
        s = lax.axis_index("s")
        pltpu.sync_copy(z_hbm.at[pl.ds(s * _RZ, _RZ)],
                        agg_sh.at[pl.ds(s * _RZ, _RZ)])
        plsc.subcore_barrier()

        g0 = c * _GC + s * 1562 + jnp.minimum(s, 8)
        ng = jnp.where(s < 8, 1563, 1562)

        def fire_g(ch, src_v, rows_v, sem):
            r0 = g0 + ch * _CG
            pltpu.sync_copy(srcp_hbm.at[pl.ds(r0, _CG)], src_v)

            @plsc.parallel_loop(0, _CG, 1, unroll=4)
            def _(j):
                pltpu.async_copy(tab_hbm.at[src_v.at[j]],
                                 rows_v.at[pl.ds(j * 128, 128)], sem)

        def wait_bytes(rows_v, sem):
            pltpu.make_async_copy(z_hbm.at[pl.ds(0, _CG * 128)], rows_v,
                                  sem).wait()

        def issue_s(ch, dst_v, rows_v, sem):
            r0 = g0 + ch * _CG
            pltpu.sync_copy(dstp_hbm.at[pl.ds(r0, _CG)], dst_v)

            @plsc.parallel_loop(0, _CG, 1, unroll=4)
            def _(j):
                pltpu.async_copy(rows_v.at[pl.ds(j * 128, 128)],
                                 agg_sh.at[dst_v.at[j]], sem, add=True)

        fire_g(0, src_a, rows_a, gsem_a)

        def body(ch, carry):
            even = lax.rem(ch, 2) == 0

            def half(src_v, dst_v, rows_v, gsem, ssem,
                     o_src, o_dst, o_rows, o_gsem, o_ssem):
                wait_bytes(rows_v, gsem)

                @pl.when(ch + 1 < _M)
                def _():
                    @pl.when(ch > 0)
                    def _():
                        wait_bytes(o_rows, o_ssem)

                    fire_g(ch + 1, o_src, o_rows, o_gsem)

                issue_s(ch, dst_v, rows_v, ssem)

            @pl.when(even)
            def _():
                half(src_a, dst_a, rows_a, gsem_a, ssem_a,
                     src_b, dst_b, rows_b, gsem_b, ssem_b)

            @pl.when(jnp.logical_not(even))
            def _():
                half(src_b, dst_b, rows_b, gsem_b, ssem_b,
                     src_a, dst_a, rows_a, gsem_a, ssem_a)

            return carry

        lax.fori_loop(0, _M, body, 0)
        wait_bytes(rows_b, ssem_b)
        wait_bytes(rows_a, ssem_a)

        def tail(g, carry):
            pltpu.sync_copy(srcp_hbm.at[pl.ds(g, 1)], src_a.at[pl.ds(0, 1)])
            pltpu.async_copy(tab_hbm.at[src_a.at[0]],
                             rows_a.at[pl.ds(0, 128)], gsem_a).wait()
            pltpu.sync_copy(dstp_hbm.at[pl.ds(g, 1)], dst_a.at[pl.ds(0, 1)])
            pltpu.sync_copy(rows_a.at[pl.ds(0, 128)],
                            agg_sh.at[dst_a.at[0]], add=True)
            return carry

        lax.fori_loop(g0 + _M * _CG, g0 + ng, tail, 0)
        plsc.subcore_barrier()
        pltpu.sync_copy(agg_sh.at[pl.ds(s * _RZ, _RZ)],
                        out_hbm.at[pl.ds(c * _NP + s * _RZ, _RZ)])

    return pass_fn


_edge_pass8 = _make_edge_pass(8)

_BR = 8192
_GRID = (_N + _BR - 1) // _BR


def _norm(d):
    return jnp.where(d > 0, lax.rsqrt(jnp.maximum(d, 1.0)), 0.0)


def _tcn_body(dp_ref, n_ref):
    n_ref[0:1] = _norm(jnp.sum(dp_ref[0:16], axis=0))[None]
    n_ref[1:2] = _norm(jnp.sum(dp_ref[16:32], axis=0))[None]


_tcn = pl.pallas_call(
    _tcn_body,
    grid=(_GRID,),
    in_specs=[pl.BlockSpec((32, _BR), lambda i: (0, i))],
    out_specs=pl.BlockSpec((2, _BR), lambda i: (0, i)),
    out_shape=jax.ShapeDtypeStruct((2, _NP), jnp.float32),
)


def _tc1a_body(x_ref, w1_ref, xw_ref):
    xw_ref[...] = jnp.dot(x_ref[...], w1_ref[...],
                          preferred_element_type=jnp.float32,
                          precision=lax.Precision.HIGHEST)


_tc1a = pl.pallas_call(
    _tc1a_body,
    grid=(_GRID,),
    in_specs=[
        pl.BlockSpec((_BR, 128), lambda i: (i, 0)),
        pl.BlockSpec((128, 8), lambda i: (0, 0)),
    ],
    out_specs=pl.BlockSpec((_BR, 8), lambda i: (i, 0)),
    out_shape=jax.ShapeDtypeStruct((_N, 8), jnp.float32),
)


def _tc1b_body(xw_ref, n_ref, h_ref):
    h_ref[...] = xw_ref[...] * n_ref[0][:, None]


_tc1b = pl.pallas_call(
    _tc1b_body,
    grid=(_GRID,),
    in_specs=[
        pl.BlockSpec((_BR, 8), lambda i: (i, 0)),
        pl.BlockSpec((2, _BR), lambda i: (0, i)),
    ],
    out_specs=pl.BlockSpec((_BR, 8), lambda i: (i, 0)),
    out_shape=jax.ShapeDtypeStruct((_NP, 8), jnp.float32),
)


def _tc2_body(aggp_ref, n_ref, b1_ref, u_ref):
    agg = aggp_ref[0] + aggp_ref[1]
    g = jnp.maximum(agg * n_ref[1][:, None] + b1_ref[...], 0.0)
    u_ref[...] = g * n_ref[0][:, None]


_tc2 = pl.pallas_call(
    _tc2_body,
    grid=(_GRID,),
    in_specs=[
        pl.BlockSpec((2, _BR, 8), lambda i: (0, i, 0)),
        pl.BlockSpec((2, _BR), lambda i: (0, i)),
        pl.BlockSpec((1, 8), lambda i: (0, 0)),
    ],
    out_specs=pl.BlockSpec((_BR, 8), lambda i: (i, 0)),
    out_shape=jax.ShapeDtypeStruct((_NP, 8), jnp.float32),
)


def _tc3_body(aggp_ref, n_ref, w2_ref, b2_ref, o_ref):
    agg = aggp_ref[0] + aggp_ref[1]
    o_ref[...] = jnp.dot(agg, w2_ref[...],
                         preferred_element_type=jnp.float32,
                         precision=lax.Precision.HIGHEST) * n_ref[1][:, None] + b2_ref[...]


_tc3 = pl.pallas_call(
    _tc3_body,
    grid=(_GRID,),
    in_specs=[
        pl.BlockSpec((2, _BR, 8), lambda i: (0, i, 0)),
        pl.BlockSpec((2, _BR), lambda i: (0, i)),
        pl.BlockSpec((8, 1), lambda i: (0, 0)),
        pl.BlockSpec((1, 1), lambda i: (0, 0)),
    ],
    out_specs=pl.BlockSpec((_BR, 1), lambda i: (i, 0)),
    out_shape=jax.ShapeDtypeStruct((_N, 1), jnp.float32),
)


def kernel(x, edge_index, W1, b1, W2, b2):
    ei = edge_index.astype(jnp.int32)
    ef = ei.reshape(-1)
    zf = jnp.zeros((_NP * 8,), jnp.float32)
    degp = _degrees_sc(ef, zf).reshape(32, _N)

    srcp = ei[0].reshape(_EPR, 128)
    dstp = ei[1].reshape(_EPR, 128)
    z8 = zf.reshape(_NP, 8)

    xw = _tc1a(x, W1)
    nrm = _tcn(degp)
    hp = _tc1b(xw, nrm)
    aggp = _edge_pass8(hp, srcp, dstp, z8).reshape(2, _NP, 8)

    up = _tc2(aggp, nrm, b1.reshape(1, 8))
    agg2p = _edge_pass8(up, srcp, dstp, z8).reshape(2, _NP, 8)

    return _tc3(agg2p, nrm, W2, b2.reshape(1, 1))
